# baseline (speedup 1.0000x reference)
import functools

import jax
import jax.numpy as jnp
from jax import lax
from jax.experimental import pallas as pl
from jax.experimental.pallas import tpu as pltpu
from jax.experimental.pallas import tpu_sc as plsc

N = 50000
NPAD = 50176
E = 800000
EPAD = 802816
VOCAB = 1340
VPAD = 1344
SD = 32
HID = 64
OUT = 41
NG = 128

EPT16 = EPAD // 16
EPT32 = EPAD // 32
NCHUNK = 392
NBUF = 8
NGRP = NCHUNK // NBUF
NPT = NPAD // 32
QW = HID // 4
RPT = NPAD // 16
BLK = 112


def _mesh():
    return plsc.VectorSubcoreMesh(core_axis_name="c", subcore_axis_name="s")


def _sc_params():
    return pltpu.CompilerParams(needs_layout_passes=False,
                                use_tc_tiling_on_sc=False)


def _sc_deg(dst, zflat):
    @functools.partial(
        pl.kernel,
        out_type=jax.ShapeDtypeStruct((32, NPAD), jnp.float32),
        mesh=_mesh(),
        compiler_params=_sc_params(),
        scratch_types=[
            pltpu.VMEM((EPT32,), jnp.int32),
            pltpu.VMEM((NPAD,), jnp.float32),
        ],
    )
    def k(dst_hbm, zflat_hbm, out_hbm, dst_v, deg_v):
        c = lax.axis_index("c")
        s = lax.axis_index("s")
        t = s * 2 + c
        for i in range(6):
            pltpu.sync_copy(zflat_hbm, deg_v.at[pl.ds(i * 8192, 8192)])
        pltpu.sync_copy(zflat_hbm.at[pl.ds(0, 1024)], deg_v.at[pl.ds(49152, 1024)])
        pltpu.sync_copy(dst_hbm.at[pl.ds(t * EPT32, EPT32)], dst_v)
        ones = jnp.ones((16,), jnp.float32)

        def body(i, carry):
            d16 = dst_v[pl.ds(i * 16, 16)]
            plsc.addupdate_scatter(deg_v, [d16], ones)
            return carry

        lax.fori_loop(0, EPT32 // 16, body, 0)
        pltpu.sync_copy(deg_v, out_hbm.at[t])

    return k(dst, zflat)


def _sc_net(edges4, ewq, x_pad, dinv16, b1q):
    @functools.partial(
        pl.kernel,
        out_type=[jax.ShapeDtypeStruct((NPAD, QW), jnp.float32)] * 12,
        mesh=_mesh(),
        compiler_params=_sc_params(),
        scratch_types=[
        ] + [pltpu.VMEM((2, 128), jnp.int32)] * NBUF
          + [pltpu.VMEM((128, QW), jnp.float32)] * NBUF + [
            pltpu.VMEM((128, QW), jnp.float32),
            pltpu.VMEM((BLK, QW), jnp.float32),
            pltpu.VMEM((BLK, QW), jnp.float32),
            pltpu.VMEM((BLK, QW), jnp.float32),
            pltpu.VMEM((BLK, QW), jnp.float32),
            pltpu.VMEM((4, QW), jnp.float32),
            pltpu.VMEM((RPT,), jnp.int32),
            pltpu.VMEM_SHARED((NPAD, QW), jnp.float32),
            pltpu.SemaphoreType.DMA,
            pltpu.SemaphoreType.DMA,
            pltpu.SemaphoreType.DMA,
            pltpu.SemaphoreType.DMA,
            pltpu.SemaphoreType.DMA,
            pltpu.SemaphoreType.DMA,
        ],
    )
    def k(edg_hbm, ew0, ew1, ew2, ew3, x_hbm, dinv_hbm, b1_hbm,
          g0, g1, g2, g3, q0, q1, q2, q3, p0, p1, p2, p3,
          *rest):
        ebs = rest[:NBUF]
        rws = rest[NBUF:2 * NBUF]
        (zbuf_v, work_v, gbuf_v, dbuf_v, qbuf_v, b1_v, x_v, acc_sh,
         semiA, semgA, semscA, semiB, semgB, semscB) = rest[2 * NBUF:]
        c = lax.axis_index("c")
        s = lax.axis_index("s")
        pltpu.sync_copy(b1_hbm, b1_v)
        pltpu.sync_copy(x_hbm.at[pl.ds(s * RPT, RPT)], x_v)

        def zrow(j, carry):
            zbuf_v[j] = jnp.zeros((QW,), jnp.float32)
            return carry
        lax.fori_loop(0, 128, zrow, 0)

        def zero_acc():
            def zblk(j, carry):
                pltpu.sync_copy(zbuf_v, acc_sh.at[pl.ds(s * RPT + j * 128, 128)])
                return carry
            lax.fori_loop(0, 24, zblk, 0)
            pltpu.sync_copy(zbuf_v.at[pl.ds(0, 64)],
                            acc_sh.at[pl.ds(s * RPT + 3072, 64)])

        def scatter_pass(h_ref):
            for b in range(NBUF):
                pltpu.async_copy(edg_hbm.at[s, b], ebs[b], semiA)

            def group(i, prefetch):
                for b in range(NBUF):
                    pltpu.make_async_copy(edg_hbm.at[s, 0], ebs[b],
                                          semiA).wait()
                for b in range(NBUF):
                    pltpu.async_copy(h_ref.at[ebs[b].at[0]], rws[b], semgA)
                for b in range(NBUF):
                    pltpu.make_async_copy(h_ref.at[ebs[b].at[0]], rws[b],
                                          semgA).wait()
                for b in range(NBUF):
                    pltpu.async_copy(rws[b], acc_sh.at[ebs[b].at[1]], semscA,
                                     add=True)
                for b in range(NBUF):
                    pltpu.make_async_copy(rws[b], acc_sh.at[ebs[b].at[1]],
                                          semscA).wait()
                if prefetch:
                    for b in range(NBUF):
                        pltpu.async_copy(edg_hbm.at[s, i * NBUF + b + NBUF],
                                         ebs[b], semiA)

            def body(i, carry):
                group(i, True)
                return carry
            lax.fori_loop(0, NGRP - 1, body, 0)
            group(NGRP - 1, False)

        def build_g(ew_ref, g_ref):
            def gblk(blk, carry):
                row = s * RPT + blk * BLK
                pltpu.async_copy(
                    ew_ref.at[x_v.at[pl.ds(blk * BLK, BLK)]], gbuf_v, semgA
                ).wait()
                pltpu.sync_copy(dinv_hbm.at[pl.ds(row, BLK)], dbuf_v)

                def grow_fn(r, c2):
                    qbuf_v[r] = gbuf_v[r] * dbuf_v[r]
                    return c2
                lax.fori_loop(0, BLK, grow_fn, 0)
                pltpu.sync_copy(qbuf_v, g_ref.at[pl.ds(row, BLK)])
                return carry
            lax.fori_loop(0, RPT // BLK, gblk, 0)

        def one_quarter(ew_ref, g_ref, q_ref, p_ref, bidx):
            b1j = b1_v[bidx]
            zero_acc()
            build_g(ew_ref, g_ref)
            plsc.subcore_barrier()
            scatter_pass(g_ref)
            plsc.subcore_barrier()

            def qblk(blk, carry):
                row = s * RPT + blk * BLK
                lrow = s * RPT + blk * BLK
                pltpu.sync_copy(acc_sh.at[pl.ds(lrow, BLK)], work_v)
                pltpu.sync_copy(g_ref.at[pl.ds(row, BLK)], gbuf_v)
                pltpu.sync_copy(dinv_hbm.at[pl.ds(row, BLK)], dbuf_v)

                def qrow(r, c2):
                    d = dbuf_v[r]
                    h = d * (work_v[r] + gbuf_v[r]) + b1j
                    qbuf_v[r] = d * jnp.maximum(h, 0.0)
                    return c2
                lax.fori_loop(0, BLK, qrow, 0)
                pltpu.sync_copy(qbuf_v, q_ref.at[pl.ds(row, BLK)])
                return carry
            lax.fori_loop(0, RPT // BLK, qblk, 0)
            plsc.subcore_barrier()
            zero_acc()
            plsc.subcore_barrier()
            scatter_pass(q_ref)
            plsc.subcore_barrier()
            pltpu.sync_copy(acc_sh.at[pl.ds(s * RPT, RPT)],
                            p_ref.at[pl.ds(s * RPT, RPT)])
            plsc.subcore_barrier()

        @pl.when(c == 0)
        def _():
            one_quarter(ew0, g0, q0, p0, 0)
            one_quarter(ew1, g1, q1, p1, 1)

        @pl.when(c == 1)
        def _():
            one_quarter(ew2, g2, q2, p2, 2)
            one_quarter(ew3, g3, q3, p3, 3)

    outs = k(edges4, *ewq, x_pad, dinv16, b1q)
    return outs[4:8], outs[8:]


def _tc_embw1(emb_pad, W1):
    def k(e_ref, w_ref, o0, o1, o2, o3):
        r = jnp.dot(e_ref[...], w_ref[...], preferred_element_type=jnp.float32)
        for i, o in enumerate((o0, o1, o2, o3)):
            o[...] = r[:, i * QW:(i + 1) * QW]
    return pl.pallas_call(
        k, out_shape=[jax.ShapeDtypeStruct((VPAD, QW), jnp.float32)] * 4,
    )(emb_pad, W1)


def _tc_dinv(parts):
    def k(p_ref, o_ref):
        i = pl.program_id(0)
        ssum = jnp.sum(p_ref[...], axis=0, keepdims=True)
        d = lax.rsqrt(ssum + 1.0)
        dcol = jnp.broadcast_to(jnp.transpose(d), (128, QW))
        rows = i * 128 + lax.broadcasted_iota(jnp.int32, (128, QW), 0)
        o_ref[...] = jnp.where(rows < N, dcol, 0.0)
    return pl.pallas_call(
        k,
        grid=(NPAD // 128,),
        in_specs=[pl.BlockSpec((32, 128), lambda i: (0, i))],
        out_specs=pl.BlockSpec((128, QW), lambda i: (i, 0)),
        out_shape=jax.ShapeDtypeStruct((NPAD, QW), jnp.float32),
    )(parts)


_BS = 1024


def _row_spec():
    return pl.BlockSpec((_BS, HID), lambda i: (i, 0))


def _q_spec():
    return pl.BlockSpec((_BS, QW), lambda i: (i, 0))


def _tc_pool(dinv16, pq, qq, batch2d, W2, b2_2d):
    BS = 512
    NBLK = NPAD // BS

    def k(d_ref, p0, p1, p2, p3, q0, q1, q2, q3, b_ref, w_ref, bias_ref,
          acc_ref, cnt_ref, out_ref):
        i = pl.program_id(0)

        @pl.when(i == 0)
        def _():
            acc_ref[...] = jnp.zeros_like(acc_ref)
            cnt_ref[...] = jnp.zeros_like(cnt_ref)

        d = d_ref[...]
        s_blk = jnp.concatenate(
            [d * (p[...] + q[...]) for p, q in
             ((p0, q0), (p1, q1), (p2, q2), (p3, q3))], axis=1)
        gid = lax.broadcasted_iota(jnp.int32, (BS, NG), 1)
        ind = jnp.where(b_ref[...] == gid, 1.0, 0.0)
        acc_ref[...] += lax.dot_general(
            ind, s_blk, (((0,), (0,)), ((), ())),
            preferred_element_type=jnp.float32)
        cnt_ref[...] += jnp.sum(ind, axis=0, keepdims=True)

        @pl.when(i == NBLK - 1)
        def _():
            out_ref[...] = (
                jnp.dot(acc_ref[...], w_ref[...],
                        preferred_element_type=jnp.float32)
                + jnp.transpose(cnt_ref[...]) * bias_ref[...])

    qspec = pl.BlockSpec((BS, QW), lambda i: (i, 0))
    outs = pl.pallas_call(
        k, grid=(NBLK,),
        in_specs=[qspec] * 9 +
                 [pl.BlockSpec((BS, 1), lambda i: (i, 0)),
                  pl.BlockSpec((HID, OUT), lambda i: (0, 0)),
                  pl.BlockSpec((1, OUT), lambda i: (0, 0))],
        out_specs=[pl.BlockSpec((NG, HID), lambda i: (0, 0)),
                   pl.BlockSpec((1, NG), lambda i: (0, 0)),
                   pl.BlockSpec((NG, OUT), lambda i: (0, 0))],
        out_shape=[jax.ShapeDtypeStruct((NG, HID), jnp.float32),
                   jax.ShapeDtypeStruct((1, NG), jnp.float32),
                   jax.ShapeDtypeStruct((NG, OUT), jnp.float32)],
    )(dinv16, *pq, *qq, batch2d, W2, b2_2d)
    return outs[2]


def kernel(x, edge_index, batch, emb, W1, b1, W2, b2):
    i32 = jnp.int32
    f32 = jnp.float32
    pad_e = jnp.full((EPAD - E,), N, i32)
    src = jnp.concatenate([edge_index[0], pad_e])
    dst = jnp.concatenate([edge_index[1], pad_e])
    edges4 = jnp.stack([src.reshape(16, NCHUNK, 128),
                        dst.reshape(16, NCHUNK, 128)], axis=2)
    x_pad = jnp.concatenate([x, jnp.zeros((NPAD - N,), i32)])
    batch2d = jnp.concatenate([batch, jnp.full((NPAD - N,), NG, i32)])[:, None]
    emb_pad = jnp.pad(emb, ((0, VPAD - VOCAB), (0, 0)))
    zflat = jnp.zeros((8192,), f32)
    b1q = b1.reshape(4, QW)
    b2_2d = b2[None, :]

    ewq = _tc_embw1(emb_pad, W1)
    parts = _sc_deg(dst, zflat)
    dinv16 = _tc_dinv(parts)
    qq, p2q = _sc_net(edges4, ewq, x_pad, dinv16, b1q)
    return _tc_pool(dinv16, p2q, qq, batch2d, W2, b2_2d)


# speedup vs baseline: 1.0004x; 1.0004x over previous
import functools

import jax
import jax.numpy as jnp
from jax import lax
from jax.experimental import pallas as pl
from jax.experimental.pallas import tpu as pltpu
from jax.experimental.pallas import tpu_sc as plsc

N = 50000
NPAD = 50176
E = 800000
EPAD = 802816
VOCAB = 1340
VPAD = 1344
SD = 32
HID = 64
OUT = 41
NG = 128

EPT16 = EPAD // 16
EPT32 = EPAD // 32
NCHUNK = 392
NBUF = 8
NGRP = NCHUNK // NBUF
NPT = NPAD // 32
QW = HID // 4
RPT = NPAD // 16
BLK = 112


def _mesh():
    return plsc.VectorSubcoreMesh(core_axis_name="c", subcore_axis_name="s")


def _sc_params():
    return pltpu.CompilerParams(needs_layout_passes=False,
                                use_tc_tiling_on_sc=False)


def _sc_deg(dst, zflat):
    @functools.partial(
        pl.kernel,
        out_type=jax.ShapeDtypeStruct((32, NPAD), jnp.float32),
        mesh=_mesh(),
        compiler_params=_sc_params(),
        scratch_types=[
            pltpu.VMEM((EPT32,), jnp.int32),
            pltpu.VMEM((NPAD,), jnp.float32),
        ],
    )
    def k(dst_hbm, zflat_hbm, out_hbm, dst_v, deg_v):
        c = lax.axis_index("c")
        s = lax.axis_index("s")
        t = s * 2 + c
        for i in range(6):
            pltpu.sync_copy(zflat_hbm, deg_v.at[pl.ds(i * 8192, 8192)])
        pltpu.sync_copy(zflat_hbm.at[pl.ds(0, 1024)], deg_v.at[pl.ds(49152, 1024)])
        pltpu.sync_copy(dst_hbm.at[pl.ds(t * EPT32, EPT32)], dst_v)
        ones = jnp.ones((16,), jnp.float32)

        def body(i, carry):
            d16 = dst_v[pl.ds(i * 16, 16)]
            plsc.addupdate_scatter(deg_v, [d16], ones)
            return carry

        lax.fori_loop(0, EPT32 // 16, body, 0)
        pltpu.sync_copy(deg_v, out_hbm.at[t])

    return k(dst, zflat)


def _sc_net(edges4, ewq, x_pad, dinv16, b1q):
    @functools.partial(
        pl.kernel,
        out_type=[jax.ShapeDtypeStruct((NPAD, QW), jnp.float32)] * 12,
        mesh=_mesh(),
        compiler_params=_sc_params(),
        scratch_types=[
        ] + [pltpu.VMEM((2, 128), jnp.int32)] * NBUF
          + [pltpu.VMEM((128, QW), jnp.float32)] * NBUF + [
            pltpu.VMEM((128, QW), jnp.float32),
            pltpu.VMEM((BLK, QW), jnp.float32),
            pltpu.VMEM((BLK, QW), jnp.float32),
            pltpu.VMEM((BLK, QW), jnp.float32),
            pltpu.VMEM((BLK, QW), jnp.float32),
            pltpu.VMEM((4, QW), jnp.float32),
            pltpu.VMEM((RPT,), jnp.int32),
            pltpu.VMEM_SHARED((NPAD, QW), jnp.float32),
            pltpu.SemaphoreType.DMA,
            pltpu.SemaphoreType.DMA,
            pltpu.SemaphoreType.DMA,
            pltpu.SemaphoreType.DMA,
            pltpu.SemaphoreType.DMA,
            pltpu.SemaphoreType.DMA,
        ],
    )
    def k(edg_hbm, ew0, ew1, ew2, ew3, x_hbm, dinv_hbm, b1_hbm,
          g0, g1, g2, g3, q0, q1, q2, q3, p0, p1, p2, p3,
          *rest):
        ebs = rest[:NBUF]
        rws = rest[NBUF:2 * NBUF]
        (zbuf_v, work_v, gbuf_v, dbuf_v, qbuf_v, b1_v, x_v, acc_sh,
         semiA, semgA, semscA, semiB, semgB, semscB) = rest[2 * NBUF:]
        c = lax.axis_index("c")
        s = lax.axis_index("s")
        pltpu.sync_copy(b1_hbm, b1_v)
        pltpu.sync_copy(x_hbm.at[pl.ds(s * RPT, RPT)], x_v)

        def zrow(j, carry):
            zbuf_v[j] = jnp.zeros((QW,), jnp.float32)
            return carry
        lax.fori_loop(0, 128, zrow, 0)

        def zero_acc():
            def zblk(j, carry):
                pltpu.sync_copy(zbuf_v, acc_sh.at[pl.ds(s * RPT + j * 128, 128)])
                return carry
            lax.fori_loop(0, 24, zblk, 0)
            pltpu.sync_copy(zbuf_v.at[pl.ds(0, 64)],
                            acc_sh.at[pl.ds(s * RPT + 3072, 64)])

        def scatter_pass(h_ref):
            for b in range(NBUF):
                pltpu.async_copy(edg_hbm.at[s, b], ebs[b], semiA)

            def group(i, prefetch):
                for b in range(NBUF):
                    pltpu.make_async_copy(edg_hbm.at[s, 0], ebs[b],
                                          semiA).wait()
                for b in range(NBUF):
                    pltpu.async_copy(h_ref.at[ebs[b].at[0]], rws[b], semgA)
                for b in range(NBUF):
                    pltpu.make_async_copy(h_ref.at[ebs[b].at[0]], rws[b],
                                          semgA).wait()
                for b in range(NBUF):
                    pltpu.async_copy(rws[b], acc_sh.at[ebs[b].at[1]], semscA,
                                     add=True)
                for b in range(NBUF):
                    pltpu.make_async_copy(rws[b], acc_sh.at[ebs[b].at[1]],
                                          semscA).wait()
                if prefetch:
                    for b in range(NBUF):
                        pltpu.async_copy(edg_hbm.at[s, i * NBUF + b + NBUF],
                                         ebs[b], semiA)

            def body(i, carry):
                group(i, True)
                return carry
            lax.fori_loop(0, NGRP - 1, body, 0)
            group(NGRP - 1, False)

        def build_g(ew_ref, g_ref):
            def gblk(blk, carry):
                row = s * RPT + blk * BLK
                pltpu.async_copy(
                    ew_ref.at[x_v.at[pl.ds(blk * BLK, BLK)]], gbuf_v, semgA
                ).wait()
                pltpu.sync_copy(dinv_hbm.at[pl.ds(row, BLK)], dbuf_v)

                def grow_fn(r, c2):
                    qbuf_v[r] = gbuf_v[r] * dbuf_v[r]
                    return c2
                lax.fori_loop(0, BLK, grow_fn, 0)
                pltpu.sync_copy(qbuf_v, g_ref.at[pl.ds(row, BLK)])
                return carry
            lax.fori_loop(0, RPT // BLK, gblk, 0)

        def one_quarter(ew_ref, g_ref, q_ref, p_ref, bidx):
            b1j = b1_v[bidx]
            zero_acc()
            build_g(ew_ref, g_ref)
            plsc.subcore_barrier()
            scatter_pass(g_ref)
            plsc.subcore_barrier()

            def qblk(blk, carry):
                row = s * RPT + blk * BLK
                pltpu.sync_copy(acc_sh.at[pl.ds(row, BLK)], work_v)
                pltpu.sync_copy(g_ref.at[pl.ds(row, BLK)], gbuf_v)
                pltpu.sync_copy(dinv_hbm.at[pl.ds(row, BLK)], dbuf_v)

                def qrow(r, c2):
                    d = dbuf_v[r]
                    h = d * (work_v[r] + gbuf_v[r]) + b1j
                    qbuf_v[r] = d * jnp.maximum(h, 0.0)
                    return c2
                lax.fori_loop(0, BLK, qrow, 0)
                pltpu.sync_copy(qbuf_v, q_ref.at[pl.ds(row, BLK)])
                return carry
            lax.fori_loop(0, RPT // BLK, qblk, 0)
            plsc.subcore_barrier()
            zero_acc()
            plsc.subcore_barrier()
            scatter_pass(q_ref)
            plsc.subcore_barrier()
            pltpu.sync_copy(acc_sh.at[pl.ds(s * RPT, RPT)],
                            p_ref.at[pl.ds(s * RPT, RPT)])
            plsc.subcore_barrier()

        @pl.when(c == 0)
        def _():
            one_quarter(ew0, g0, q0, p0, 0)
            one_quarter(ew1, g1, q1, p1, 1)

        @pl.when(c == 1)
        def _():
            one_quarter(ew2, g2, q2, p2, 2)
            one_quarter(ew3, g3, q3, p3, 3)

    outs = k(edges4, *ewq, x_pad, dinv16, b1q)
    return outs[4:8], outs[8:]


def _tc_embw1(emb_pad, W1):
    def k(e_ref, w_ref, o0, o1, o2, o3):
        r = jnp.dot(e_ref[...], w_ref[...], preferred_element_type=jnp.float32)
        for i, o in enumerate((o0, o1, o2, o3)):
            o[...] = r[:, i * QW:(i + 1) * QW]
    return pl.pallas_call(
        k, out_shape=[jax.ShapeDtypeStruct((VPAD, QW), jnp.float32)] * 4,
    )(emb_pad, W1)


def _tc_dinv(parts):
    def k(p_ref, o_ref):
        i = pl.program_id(0)
        ssum = jnp.sum(p_ref[...], axis=0, keepdims=True)
        d = lax.rsqrt(ssum + 1.0)
        dcol = jnp.broadcast_to(jnp.transpose(d), (128, QW))
        rows = i * 128 + lax.broadcasted_iota(jnp.int32, (128, QW), 0)
        o_ref[...] = jnp.where(rows < N, dcol, 0.0)
    return pl.pallas_call(
        k,
        grid=(NPAD // 128,),
        in_specs=[pl.BlockSpec((32, 128), lambda i: (0, i))],
        out_specs=pl.BlockSpec((128, QW), lambda i: (i, 0)),
        out_shape=jax.ShapeDtypeStruct((NPAD, QW), jnp.float32),
    )(parts)


_BS = 1024


def _row_spec():
    return pl.BlockSpec((_BS, HID), lambda i: (i, 0))


def _q_spec():
    return pl.BlockSpec((_BS, QW), lambda i: (i, 0))


def _tc_pool(dinv16, pq, qq, batch2d, W2, b2_2d):
    BS = 512
    NBLK = NPAD // BS

    def k(d_ref, p0, p1, p2, p3, q0, q1, q2, q3, b_ref, w_ref, bias_ref,
          acc_ref, cnt_ref, out_ref):
        i = pl.program_id(0)

        @pl.when(i == 0)
        def _():
            acc_ref[...] = jnp.zeros_like(acc_ref)
            cnt_ref[...] = jnp.zeros_like(cnt_ref)

        d = d_ref[...]
        s_blk = jnp.concatenate(
            [d * (p[...] + q[...]) for p, q in
             ((p0, q0), (p1, q1), (p2, q2), (p3, q3))], axis=1)
        gid = lax.broadcasted_iota(jnp.int32, (BS, NG), 1)
        ind = jnp.where(b_ref[...] == gid, 1.0, 0.0)
        acc_ref[...] += lax.dot_general(
            ind, s_blk, (((0,), (0,)), ((), ())),
            preferred_element_type=jnp.float32)
        cnt_ref[...] += jnp.sum(ind, axis=0, keepdims=True)

        @pl.when(i == NBLK - 1)
        def _():
            out_ref[...] = (
                jnp.dot(acc_ref[...], w_ref[...],
                        preferred_element_type=jnp.float32)
                + jnp.transpose(cnt_ref[...]) * bias_ref[...])

    qspec = pl.BlockSpec((BS, QW), lambda i: (i, 0))
    outs = pl.pallas_call(
        k, grid=(NBLK,),
        in_specs=[qspec] * 9 +
                 [pl.BlockSpec((BS, 1), lambda i: (i, 0)),
                  pl.BlockSpec((HID, OUT), lambda i: (0, 0)),
                  pl.BlockSpec((1, OUT), lambda i: (0, 0))],
        out_specs=[pl.BlockSpec((NG, HID), lambda i: (0, 0)),
                   pl.BlockSpec((1, NG), lambda i: (0, 0)),
                   pl.BlockSpec((NG, OUT), lambda i: (0, 0))],
        out_shape=[jax.ShapeDtypeStruct((NG, HID), jnp.float32),
                   jax.ShapeDtypeStruct((1, NG), jnp.float32),
                   jax.ShapeDtypeStruct((NG, OUT), jnp.float32)],
    )(dinv16, *pq, *qq, batch2d, W2, b2_2d)
    return outs[2]


def kernel(x, edge_index, batch, emb, W1, b1, W2, b2):
    i32 = jnp.int32
    f32 = jnp.float32
    pad_e = jnp.full((EPAD - E,), N, i32)
    src = jnp.concatenate([edge_index[0], pad_e])
    dst = jnp.concatenate([edge_index[1], pad_e])
    edges4 = jnp.stack([src.reshape(16, NCHUNK, 128),
                        dst.reshape(16, NCHUNK, 128)], axis=2)
    x_pad = jnp.concatenate([x, jnp.zeros((NPAD - N,), i32)])
    batch2d = jnp.concatenate([batch, jnp.full((NPAD - N,), NG, i32)])[:, None]
    emb_pad = jnp.pad(emb, ((0, VPAD - VOCAB), (0, 0)))
    zflat = jnp.zeros((8192,), f32)
    b1q = b1.reshape(4, QW)
    b2_2d = b2[None, :]

    ewq = _tc_embw1(emb_pad, W1)
    parts = _sc_deg(dst, zflat)
    dinv16 = _tc_dinv(parts)
    qq, p2q = _sc_net(edges4, ewq, x_pad, dinv16, b1q)
    return _tc_pool(dinv16, p2q, qq, batch2d, W2, b2_2d)


# speedup vs baseline: 1.0665x; 1.0660x over previous
import functools

import jax
import jax.numpy as jnp
from jax import lax
from jax.experimental import pallas as pl
from jax.experimental.pallas import tpu as pltpu
from jax.experimental.pallas import tpu_sc as plsc

N = 50000
NPAD = 50176
E = 800000
EPAD = 802816
VOCAB = 1340
VPAD = 1344
SD = 32
HID = 64
OUT = 41
NG = 128

EPT16 = EPAD // 16
EPT32 = EPAD // 32
NCHUNK = 392
NBUF = 14
NGRP = NCHUNK // NBUF
NPT = NPAD // 32
QW = HID // 4
RPT = NPAD // 16
BLK = 112


def _mesh():
    return plsc.VectorSubcoreMesh(core_axis_name="c", subcore_axis_name="s")


def _sc_params():
    return pltpu.CompilerParams(needs_layout_passes=False,
                                use_tc_tiling_on_sc=False)


def _sc_deg(dst, zflat):
    @functools.partial(
        pl.kernel,
        out_type=jax.ShapeDtypeStruct((32, NPAD), jnp.float32),
        mesh=_mesh(),
        compiler_params=_sc_params(),
        scratch_types=[
            pltpu.VMEM((EPT32,), jnp.int32),
            pltpu.VMEM((NPAD,), jnp.float32),
        ],
    )
    def k(dst_hbm, zflat_hbm, out_hbm, dst_v, deg_v):
        c = lax.axis_index("c")
        s = lax.axis_index("s")
        t = s * 2 + c
        for i in range(6):
            pltpu.sync_copy(zflat_hbm, deg_v.at[pl.ds(i * 8192, 8192)])
        pltpu.sync_copy(zflat_hbm.at[pl.ds(0, 1024)], deg_v.at[pl.ds(49152, 1024)])
        pltpu.sync_copy(dst_hbm.at[pl.ds(t * EPT32, EPT32)], dst_v)
        ones = jnp.ones((16,), jnp.float32)

        def body(i, carry):
            d16 = dst_v[pl.ds(i * 16, 16)]
            plsc.addupdate_scatter(deg_v, [d16], ones)
            return carry

        lax.fori_loop(0, EPT32 // 16, body, 0)
        pltpu.sync_copy(deg_v, out_hbm.at[t])

    return k(dst, zflat)


def _sc_net(edges4, ewq, x_pad, dinv16, b1q):
    @functools.partial(
        pl.kernel,
        out_type=[jax.ShapeDtypeStruct((NPAD, QW), jnp.float32)] * 12,
        mesh=_mesh(),
        compiler_params=_sc_params(),
        scratch_types=[
        ] + [pltpu.VMEM((2, 128), jnp.int32)] * NBUF
          + [pltpu.VMEM((128, QW), jnp.float32)] * NBUF + [
            pltpu.VMEM((128, QW), jnp.float32),
            pltpu.VMEM((BLK, QW), jnp.float32),
            pltpu.VMEM((BLK, QW), jnp.float32),
            pltpu.VMEM((BLK, QW), jnp.float32),
            pltpu.VMEM((BLK, QW), jnp.float32),
            pltpu.VMEM((4, QW), jnp.float32),
            pltpu.VMEM((RPT,), jnp.int32),
            pltpu.VMEM_SHARED((NPAD, QW), jnp.float32),
            pltpu.SemaphoreType.DMA,
            pltpu.SemaphoreType.DMA,
            pltpu.SemaphoreType.DMA,
            pltpu.SemaphoreType.DMA,
            pltpu.SemaphoreType.DMA,
            pltpu.SemaphoreType.DMA,
        ],
    )
    def k(edg_hbm, ew0, ew1, ew2, ew3, x_hbm, dinv_hbm, b1_hbm,
          g0, g1, g2, g3, q0, q1, q2, q3, p0, p1, p2, p3,
          *rest):
        ebs = rest[:NBUF]
        rws = rest[NBUF:2 * NBUF]
        (zbuf_v, work_v, gbuf_v, dbuf_v, qbuf_v, b1_v, x_v, acc_sh,
         semiA, semgA, semscA, semiB, semgB, semscB) = rest[2 * NBUF:]
        c = lax.axis_index("c")
        s = lax.axis_index("s")
        pltpu.sync_copy(b1_hbm, b1_v)
        pltpu.sync_copy(x_hbm.at[pl.ds(s * RPT, RPT)], x_v)

        def zrow(j, carry):
            zbuf_v[j] = jnp.zeros((QW,), jnp.float32)
            return carry
        lax.fori_loop(0, 128, zrow, 0)

        def zero_acc():
            def zblk(j, carry):
                pltpu.sync_copy(zbuf_v, acc_sh.at[pl.ds(s * RPT + j * 128, 128)])
                return carry
            lax.fori_loop(0, 24, zblk, 0)
            pltpu.sync_copy(zbuf_v.at[pl.ds(0, 64)],
                            acc_sh.at[pl.ds(s * RPT + 3072, 64)])

        def scatter_pass(h_ref):
            for b in range(NBUF):
                pltpu.async_copy(edg_hbm.at[s, b], ebs[b], semiA)

            def group(i, prefetch):
                for b in range(NBUF):
                    pltpu.make_async_copy(edg_hbm.at[s, 0], ebs[b],
                                          semiA).wait()
                for b in range(NBUF):
                    pltpu.async_copy(h_ref.at[ebs[b].at[0]], rws[b], semgA)
                for b in range(NBUF):
                    pltpu.make_async_copy(h_ref.at[ebs[b].at[0]], rws[b],
                                          semgA).wait()
                for b in range(NBUF):
                    pltpu.async_copy(rws[b], acc_sh.at[ebs[b].at[1]], semscA,
                                     add=True)
                for b in range(NBUF):
                    pltpu.make_async_copy(rws[b], acc_sh.at[ebs[b].at[1]],
                                          semscA).wait()
                if prefetch:
                    for b in range(NBUF):
                        pltpu.async_copy(edg_hbm.at[s, i * NBUF + b + NBUF],
                                         ebs[b], semiA)

            def body(i, carry):
                group(i, True)
                return carry
            lax.fori_loop(0, NGRP - 1, body, 0)
            group(NGRP - 1, False)

        def build_g(ew_ref, g_ref):
            def gblk(blk, carry):
                row = s * RPT + blk * BLK
                pltpu.async_copy(
                    ew_ref.at[x_v.at[pl.ds(blk * BLK, BLK)]], gbuf_v, semgA
                ).wait()
                pltpu.sync_copy(dinv_hbm.at[pl.ds(row, BLK)], dbuf_v)

                def grow_fn(r, c2):
                    qbuf_v[r] = gbuf_v[r] * dbuf_v[r]
                    return c2
                lax.fori_loop(0, BLK, grow_fn, 0)
                pltpu.sync_copy(qbuf_v, g_ref.at[pl.ds(row, BLK)])
                return carry
            lax.fori_loop(0, RPT // BLK, gblk, 0)

        def one_quarter(ew_ref, g_ref, q_ref, p_ref, bidx):
            b1j = b1_v[bidx]
            zero_acc()
            build_g(ew_ref, g_ref)
            plsc.subcore_barrier()
            scatter_pass(g_ref)
            plsc.subcore_barrier()

            def qblk(blk, carry):
                row = s * RPT + blk * BLK
                pltpu.sync_copy(acc_sh.at[pl.ds(row, BLK)], work_v)
                pltpu.sync_copy(g_ref.at[pl.ds(row, BLK)], gbuf_v)
                pltpu.sync_copy(dinv_hbm.at[pl.ds(row, BLK)], dbuf_v)

                def qrow(r, c2):
                    d = dbuf_v[r]
                    h = d * (work_v[r] + gbuf_v[r]) + b1j
                    qbuf_v[r] = d * jnp.maximum(h, 0.0)
                    return c2
                lax.fori_loop(0, BLK, qrow, 0)
                pltpu.sync_copy(qbuf_v, q_ref.at[pl.ds(row, BLK)])
                return carry
            lax.fori_loop(0, RPT // BLK, qblk, 0)
            plsc.subcore_barrier()
            zero_acc()
            plsc.subcore_barrier()
            scatter_pass(q_ref)
            plsc.subcore_barrier()
            pltpu.sync_copy(acc_sh.at[pl.ds(s * RPT, RPT)],
                            p_ref.at[pl.ds(s * RPT, RPT)])
            plsc.subcore_barrier()

        @pl.when(c == 0)
        def _():
            one_quarter(ew0, g0, q0, p0, 0)
            one_quarter(ew1, g1, q1, p1, 1)

        @pl.when(c == 1)
        def _():
            one_quarter(ew2, g2, q2, p2, 2)
            one_quarter(ew3, g3, q3, p3, 3)

    outs = k(edges4, *ewq, x_pad, dinv16, b1q)
    return outs[4:8], outs[8:]


def _tc_embw1(emb_pad, W1):
    def k(e_ref, w_ref, o0, o1, o2, o3):
        r = jnp.dot(e_ref[...], w_ref[...], preferred_element_type=jnp.float32)
        for i, o in enumerate((o0, o1, o2, o3)):
            o[...] = r[:, i * QW:(i + 1) * QW]
    return pl.pallas_call(
        k, out_shape=[jax.ShapeDtypeStruct((VPAD, QW), jnp.float32)] * 4,
    )(emb_pad, W1)


def _tc_dinv(parts):
    def k(p_ref, o_ref):
        i = pl.program_id(0)
        ssum = jnp.sum(p_ref[...], axis=0, keepdims=True)
        d = lax.rsqrt(ssum + 1.0)
        dcol = jnp.broadcast_to(jnp.transpose(d), (128, QW))
        rows = i * 128 + lax.broadcasted_iota(jnp.int32, (128, QW), 0)
        o_ref[...] = jnp.where(rows < N, dcol, 0.0)
    return pl.pallas_call(
        k,
        grid=(NPAD // 128,),
        in_specs=[pl.BlockSpec((32, 128), lambda i: (0, i))],
        out_specs=pl.BlockSpec((128, QW), lambda i: (i, 0)),
        out_shape=jax.ShapeDtypeStruct((NPAD, QW), jnp.float32),
    )(parts)


_BS = 1024


def _row_spec():
    return pl.BlockSpec((_BS, HID), lambda i: (i, 0))


def _q_spec():
    return pl.BlockSpec((_BS, QW), lambda i: (i, 0))


def _tc_pool(dinv16, pq, qq, batch2d, W2, b2_2d):
    BS = 512
    NBLK = NPAD // BS

    def k(d_ref, p0, p1, p2, p3, q0, q1, q2, q3, b_ref, w_ref, bias_ref,
          acc_ref, cnt_ref, out_ref):
        i = pl.program_id(0)

        @pl.when(i == 0)
        def _():
            acc_ref[...] = jnp.zeros_like(acc_ref)
            cnt_ref[...] = jnp.zeros_like(cnt_ref)

        d = d_ref[...]
        s_blk = jnp.concatenate(
            [d * (p[...] + q[...]) for p, q in
             ((p0, q0), (p1, q1), (p2, q2), (p3, q3))], axis=1)
        gid = lax.broadcasted_iota(jnp.int32, (BS, NG), 1)
        ind = jnp.where(b_ref[...] == gid, 1.0, 0.0)
        acc_ref[...] += lax.dot_general(
            ind, s_blk, (((0,), (0,)), ((), ())),
            preferred_element_type=jnp.float32)
        cnt_ref[...] += jnp.sum(ind, axis=0, keepdims=True)

        @pl.when(i == NBLK - 1)
        def _():
            out_ref[...] = (
                jnp.dot(acc_ref[...], w_ref[...],
                        preferred_element_type=jnp.float32)
                + jnp.transpose(cnt_ref[...]) * bias_ref[...])

    qspec = pl.BlockSpec((BS, QW), lambda i: (i, 0))
    outs = pl.pallas_call(
        k, grid=(NBLK,),
        in_specs=[qspec] * 9 +
                 [pl.BlockSpec((BS, 1), lambda i: (i, 0)),
                  pl.BlockSpec((HID, OUT), lambda i: (0, 0)),
                  pl.BlockSpec((1, OUT), lambda i: (0, 0))],
        out_specs=[pl.BlockSpec((NG, HID), lambda i: (0, 0)),
                   pl.BlockSpec((1, NG), lambda i: (0, 0)),
                   pl.BlockSpec((NG, OUT), lambda i: (0, 0))],
        out_shape=[jax.ShapeDtypeStruct((NG, HID), jnp.float32),
                   jax.ShapeDtypeStruct((1, NG), jnp.float32),
                   jax.ShapeDtypeStruct((NG, OUT), jnp.float32)],
    )(dinv16, *pq, *qq, batch2d, W2, b2_2d)
    return outs[2]


def kernel(x, edge_index, batch, emb, W1, b1, W2, b2):
    i32 = jnp.int32
    f32 = jnp.float32
    pad_e = jnp.full((EPAD - E,), N, i32)
    src = jnp.concatenate([edge_index[0], pad_e])
    dst = jnp.concatenate([edge_index[1], pad_e])
    edges4 = jnp.stack([src.reshape(16, NCHUNK, 128),
                        dst.reshape(16, NCHUNK, 128)], axis=2)
    x_pad = jnp.concatenate([x, jnp.zeros((NPAD - N,), i32)])
    batch2d = jnp.concatenate([batch, jnp.full((NPAD - N,), NG, i32)])[:, None]
    emb_pad = jnp.pad(emb, ((0, VPAD - VOCAB), (0, 0)))
    zflat = jnp.zeros((8192,), f32)
    b1q = b1.reshape(4, QW)
    b2_2d = b2[None, :]

    ewq = _tc_embw1(emb_pad, W1)
    parts = _sc_deg(dst, zflat)
    dinv16 = _tc_dinv(parts)
    qq, p2q = _sc_net(edges4, ewq, x_pad, dinv16, b1q)
    return _tc_pool(dinv16, p2q, qq, batch2d, W2, b2_2d)


# speedup vs baseline: 1.0669x; 1.0004x over previous
import functools

import jax
import jax.numpy as jnp
from jax import lax
from jax.experimental import pallas as pl
from jax.experimental.pallas import tpu as pltpu
from jax.experimental.pallas import tpu_sc as plsc

N = 50000
NPAD = 50176
E = 800000
EPAD = 802816
VOCAB = 1340
VPAD = 1344
SD = 32
HID = 64
OUT = 41
NG = 128

EPT16 = EPAD // 16
EPT32 = EPAD // 32
NCHUNK = 392
NBUF = 14
NGRP = NCHUNK // NBUF
QW = HID // 4
RPT = NPAD // 16
BLK = 112


def _mesh():
    return plsc.VectorSubcoreMesh(core_axis_name="c", subcore_axis_name="s")


def _sc_params():
    return pltpu.CompilerParams(needs_layout_passes=False,
                                use_tc_tiling_on_sc=False)


def _sc_deg(dst, zflat):
    @functools.partial(
        pl.kernel,
        out_type=jax.ShapeDtypeStruct((32, NPAD), jnp.float32),
        mesh=_mesh(),
        compiler_params=_sc_params(),
        scratch_types=[
            pltpu.VMEM((EPT32,), jnp.int32),
            pltpu.VMEM((NPAD,), jnp.float32),
        ],
    )
    def k(dst_hbm, zflat_hbm, out_hbm, dst_v, deg_v):
        c = lax.axis_index("c")
        s = lax.axis_index("s")
        t = s * 2 + c
        for i in range(6):
            pltpu.sync_copy(zflat_hbm, deg_v.at[pl.ds(i * 8192, 8192)])
        pltpu.sync_copy(zflat_hbm.at[pl.ds(0, 1024)], deg_v.at[pl.ds(49152, 1024)])
        pltpu.sync_copy(dst_hbm.at[pl.ds(t * EPT32, EPT32)], dst_v)
        ones = jnp.ones((16,), jnp.float32)

        def body(i, carry):
            d16 = dst_v[pl.ds(i * 16, 16)]
            plsc.addupdate_scatter(deg_v, [d16], ones)
            return carry

        lax.fori_loop(0, EPT32 // 16, body, 0)
        pltpu.sync_copy(deg_v, out_hbm.at[t])

    return k(dst, zflat)


def _sc_net(edges4, ewq, x_pad, dinv16, b1q):
    @functools.partial(
        pl.kernel,
        out_type=[jax.ShapeDtypeStruct((NPAD, QW), jnp.float32)] * 12,
        mesh=_mesh(),
        compiler_params=_sc_params(),
        scratch_types=[
        ] + [pltpu.VMEM((2, 128), jnp.int32)] * NBUF
          + [pltpu.VMEM((128, QW), jnp.float32)] * NBUF + [
            pltpu.VMEM((128, QW), jnp.float32),
            pltpu.VMEM((BLK, QW), jnp.float32),
            pltpu.VMEM((BLK, QW), jnp.float32),
            pltpu.VMEM((BLK, QW), jnp.float32),
            pltpu.VMEM((BLK, QW), jnp.float32),
            pltpu.VMEM((4, QW), jnp.float32),
            pltpu.VMEM((RPT,), jnp.int32),
            pltpu.VMEM_SHARED((NPAD, QW), jnp.float32),
            pltpu.SemaphoreType.DMA,
            pltpu.SemaphoreType.DMA,
            pltpu.SemaphoreType.DMA,
            pltpu.SemaphoreType.DMA,
            pltpu.SemaphoreType.DMA,
            pltpu.SemaphoreType.DMA,
        ],
    )
    def k(edg_hbm, ew0, ew1, ew2, ew3, x_hbm, dinv_hbm, b1_hbm,
          g0, g1, g2, g3, q0, q1, q2, q3, p0, p1, p2, p3,
          *rest):
        ebs = rest[:NBUF]
        rws = rest[NBUF:2 * NBUF]
        (zbuf_v, work_v, gbuf_v, dbuf_v, qbuf_v, b1_v, x_v, acc_sh,
         semiA, semgA, semscA, semiB, semgB, semscB) = rest[2 * NBUF:]
        c = lax.axis_index("c")
        s = lax.axis_index("s")
        pltpu.sync_copy(b1_hbm, b1_v)
        pltpu.sync_copy(x_hbm.at[pl.ds(s * RPT, RPT)], x_v)

        def zrow(j, carry):
            zbuf_v[j] = jnp.zeros((QW,), jnp.float32)
            return carry
        lax.fori_loop(0, 128, zrow, 0)

        def zero_acc():
            def zblk(j, carry):
                pltpu.sync_copy(zbuf_v, acc_sh.at[pl.ds(s * RPT + j * 128, 128)])
                return carry
            lax.fori_loop(0, 24, zblk, 0)
            pltpu.sync_copy(zbuf_v.at[pl.ds(0, 64)],
                            acc_sh.at[pl.ds(s * RPT + 3072, 64)])

        def scatter_pass(h_ref):
            for b in range(NBUF):
                pltpu.async_copy(edg_hbm.at[s, b], ebs[b], semiA)

            def group(i, prefetch):
                for b in range(NBUF):
                    pltpu.make_async_copy(edg_hbm.at[s, 0], ebs[b],
                                          semiA).wait()
                for b in range(NBUF):
                    pltpu.async_copy(h_ref.at[ebs[b].at[0]], rws[b], semgA)
                for b in range(NBUF):
                    pltpu.make_async_copy(h_ref.at[ebs[b].at[0]], rws[b],
                                          semgA).wait()
                for b in range(NBUF):
                    pltpu.async_copy(rws[b], acc_sh.at[ebs[b].at[1]], semscA,
                                     add=True)
                for b in range(NBUF):
                    pltpu.make_async_copy(rws[b], acc_sh.at[ebs[b].at[1]],
                                          semscA).wait()
                if prefetch:
                    for b in range(NBUF):
                        pltpu.async_copy(edg_hbm.at[s, i * NBUF + b + NBUF],
                                         ebs[b], semiA)

            def body(i, carry):
                group(i, True)
                return carry
            lax.fori_loop(0, NGRP - 1, body, 0)
            group(NGRP - 1, False)

        def build_g(ew_ref, g_ref):
            def gblk(blk, carry):
                row = s * RPT + blk * BLK
                pltpu.async_copy(
                    ew_ref.at[x_v.at[pl.ds(blk * BLK, BLK)]], gbuf_v, semgA
                ).wait()
                pltpu.sync_copy(dinv_hbm.at[pl.ds(row, BLK)], dbuf_v)

                def grow_fn(r, c2):
                    qbuf_v[r] = gbuf_v[r] * dbuf_v[r]
                    return c2
                lax.fori_loop(0, BLK, grow_fn, 0)
                pltpu.sync_copy(qbuf_v, g_ref.at[pl.ds(row, BLK)])
                return carry
            lax.fori_loop(0, RPT // BLK, gblk, 0)

        def one_quarter(ew_ref, g_ref, q_ref, p_ref, bidx):
            b1j = b1_v[bidx]
            zero_acc()
            build_g(ew_ref, g_ref)
            plsc.subcore_barrier()
            scatter_pass(g_ref)
            plsc.subcore_barrier()

            def qblk(blk, carry):
                row = s * RPT + blk * BLK
                pltpu.sync_copy(acc_sh.at[pl.ds(row, BLK)], work_v)
                pltpu.sync_copy(g_ref.at[pl.ds(row, BLK)], gbuf_v)
                pltpu.sync_copy(dinv_hbm.at[pl.ds(row, BLK)], dbuf_v)

                def qrow(r, c2):
                    d = dbuf_v[r]
                    h = d * (work_v[r] + gbuf_v[r]) + b1j
                    qbuf_v[r] = d * jnp.maximum(h, 0.0)
                    return c2
                lax.fori_loop(0, BLK, qrow, 0)
                pltpu.sync_copy(qbuf_v, q_ref.at[pl.ds(row, BLK)])
                return carry
            lax.fori_loop(0, RPT // BLK, qblk, 0)
            plsc.subcore_barrier()
            zero_acc()
            plsc.subcore_barrier()
            scatter_pass(q_ref)
            plsc.subcore_barrier()
            pltpu.sync_copy(acc_sh.at[pl.ds(s * RPT, RPT)],
                            p_ref.at[pl.ds(s * RPT, RPT)])
            plsc.subcore_barrier()

        @pl.when(c == 0)
        def _():
            one_quarter(ew0, g0, q0, p0, 0)
            one_quarter(ew1, g1, q1, p1, 1)

        @pl.when(c == 1)
        def _():
            one_quarter(ew2, g2, q2, p2, 2)
            one_quarter(ew3, g3, q3, p3, 3)

    outs = k(edges4, *ewq, x_pad, dinv16, b1q)
    return outs[4:8], outs[8:]


def _tc_embw1(emb_pad, W1):
    def k(e_ref, w_ref, o0, o1, o2, o3):
        r = jnp.dot(e_ref[...], w_ref[...], preferred_element_type=jnp.float32)
        for i, o in enumerate((o0, o1, o2, o3)):
            o[...] = r[:, i * QW:(i + 1) * QW]
    return pl.pallas_call(
        k, out_shape=[jax.ShapeDtypeStruct((VPAD, QW), jnp.float32)] * 4,
    )(emb_pad, W1)


def _tc_dinv(parts):
    def k(p_ref, o_ref):
        i = pl.program_id(0)
        ssum = jnp.sum(p_ref[...], axis=0, keepdims=True)
        d = lax.rsqrt(ssum + 1.0)
        dcol = jnp.broadcast_to(jnp.transpose(d), (128, QW))
        rows = i * 128 + lax.broadcasted_iota(jnp.int32, (128, QW), 0)
        o_ref[...] = jnp.where(rows < N, dcol, 0.0)
    return pl.pallas_call(
        k,
        grid=(NPAD // 128,),
        in_specs=[pl.BlockSpec((32, 128), lambda i: (0, i))],
        out_specs=pl.BlockSpec((128, QW), lambda i: (i, 0)),
        out_shape=jax.ShapeDtypeStruct((NPAD, QW), jnp.float32),
    )(parts)


_BS = 1024


def _row_spec():
    return pl.BlockSpec((_BS, HID), lambda i: (i, 0))


def _q_spec():
    return pl.BlockSpec((_BS, QW), lambda i: (i, 0))


def _tc_pool(dinv16, pq, qq, batch2d, W2, b2_2d):
    BS = 512
    NBLK = NPAD // BS

    def k(d_ref, p0, p1, p2, p3, q0, q1, q2, q3, b_ref, w_ref, bias_ref,
          acc_ref, cnt_ref, out_ref):
        i = pl.program_id(0)

        @pl.when(i == 0)
        def _():
            acc_ref[...] = jnp.zeros_like(acc_ref)
            cnt_ref[...] = jnp.zeros_like(cnt_ref)

        d = d_ref[...]
        s_blk = jnp.concatenate(
            [d * (p[...] + q[...]) for p, q in
             ((p0, q0), (p1, q1), (p2, q2), (p3, q3))], axis=1)
        gid = lax.broadcasted_iota(jnp.int32, (BS, NG), 1)
        ind = jnp.where(b_ref[...] == gid, 1.0, 0.0)
        acc_ref[...] += lax.dot_general(
            ind, s_blk, (((0,), (0,)), ((), ())),
            preferred_element_type=jnp.float32)
        cnt_ref[...] += jnp.sum(ind, axis=0, keepdims=True)

        @pl.when(i == NBLK - 1)
        def _():
            out_ref[...] = (
                jnp.dot(acc_ref[...], w_ref[...],
                        preferred_element_type=jnp.float32)
                + jnp.transpose(cnt_ref[...]) * bias_ref[...])

    qspec = pl.BlockSpec((BS, QW), lambda i: (i, 0))
    outs = pl.pallas_call(
        k, grid=(NBLK,),
        in_specs=[qspec] * 9 +
                 [pl.BlockSpec((BS, 1), lambda i: (i, 0)),
                  pl.BlockSpec((HID, OUT), lambda i: (0, 0)),
                  pl.BlockSpec((1, OUT), lambda i: (0, 0))],
        out_specs=[pl.BlockSpec((NG, HID), lambda i: (0, 0)),
                   pl.BlockSpec((1, NG), lambda i: (0, 0)),
                   pl.BlockSpec((NG, OUT), lambda i: (0, 0))],
        out_shape=[jax.ShapeDtypeStruct((NG, HID), jnp.float32),
                   jax.ShapeDtypeStruct((1, NG), jnp.float32),
                   jax.ShapeDtypeStruct((NG, OUT), jnp.float32)],
    )(dinv16, *pq, *qq, batch2d, W2, b2_2d)
    return outs[2]


def kernel(x, edge_index, batch, emb, W1, b1, W2, b2):
    i32 = jnp.int32
    f32 = jnp.float32
    pad_e = jnp.full((EPAD - E,), N, i32)
    src = jnp.concatenate([edge_index[0], pad_e])
    dst = jnp.concatenate([edge_index[1], pad_e])
    edges4 = jnp.stack([src.reshape(16, NCHUNK, 128),
                        dst.reshape(16, NCHUNK, 128)], axis=2)
    x_pad = jnp.concatenate([x, jnp.zeros((NPAD - N,), i32)])
    batch2d = jnp.concatenate([batch, jnp.full((NPAD - N,), NG, i32)])[:, None]
    emb_pad = jnp.pad(emb, ((0, VPAD - VOCAB), (0, 0)))
    zflat = jnp.zeros((8192,), f32)
    b1q = b1.reshape(4, QW)
    b2_2d = b2[None, :]

    ewq = _tc_embw1(emb_pad, W1)
    parts = _sc_deg(dst, zflat)
    dinv16 = _tc_dinv(parts)
    qq, p2q = _sc_net(edges4, ewq, x_pad, dinv16, b1q)
    return _tc_pool(dinv16, p2q, qq, batch2d, W2, b2_2d)
